# baseline (speedup 1.0000x reference)
import functools

import jax
import jax.numpy as jnp
import numpy as np
from jax.experimental import pallas as pl
from jax.experimental.pallas import tpu as pltpu

LANES = 128


def _roll(v, shift, axis):
    return pltpu.roll(v, shift % v.shape[axis], axis)


def _flip(w, m):
    return jax.lax.bitcast_convert_type(
        jax.lax.bitcast_convert_type(w, jnp.int32) ^ m, jnp.float32)


def _lane_roll_mxu(w, pmat):
    b, o, i, l = w.shape
    w2 = w.reshape(b * o * i, l)
    r = jax.lax.dot_general(w2, pmat, (((1,), (0,)), ((), ())),
                            preferred_element_type=jnp.float32,
                            precision=jax.lax.Precision.HIGHEST)
    return r.reshape(b, o, i, l)


def _cmp_exchange_asc(w, j, pm_ref):
    outer, inner = w.shape[1], w.shape[2]
    grps = outer * inner
    if j < inner:
        axis, sh = 2, j
        it = jax.lax.broadcasted_iota(jnp.int32, (1, 1, inner, 1), 2)
    elif j < grps:
        axis, sh = 1, j // inner
        it = jax.lax.broadcasted_iota(jnp.int32, (1, outer, 1, 1), 1)
    else:
        sh = j // grps
        it = jax.lax.broadcasted_iota(jnp.int32, (1, 1, 1, LANES), 3)
        left = (it & sh) == 0
        s = sh.bit_length() - 1
        b = _lane_roll_mxu(w, pm_ref[2 * s])
        mn = jnp.minimum(w, b)
        mx = jnp.maximum(w, b)
        mxr = _lane_roll_mxu(mx, pm_ref[2 * s + 1])
        return jnp.where(left, mn, mxr)
    left = (it & sh) == 0
    b = _roll(w, -sh, axis)
    mn = jnp.minimum(w, b)
    mx = jnp.maximum(w, b)
    mxr = _roll(mx, sh, axis)
    return jnp.where(left, mn, mxr)


def _flip_masks(grps):
    n = grps * LANES
    log_n = n.bit_length() - 1
    li = np.arange(LANES, dtype=np.int64)[None, :]
    gi = np.arange(grps, dtype=np.int64)[:, None]
    idxn = li * grps + gi
    neg = np.int32(-2 ** 31)

    def smask(k):
        return np.where((idxn & k) != 0, neg, np.int32(0)).astype(np.int32)

    out = [smask(2)]
    for stage in range(1, log_n):
        k = 1 << stage
        out.append(smask(k) ^ smask(2 * k))
    inner = min(grps, 8)
    return np.stack(out).reshape(log_n, grps // inner, inner, LANES)


def _perm_mats(grps, n):
    n_steps = (n // grps).bit_length() - 1
    eye = np.eye(LANES, dtype=np.float32)
    mats = []
    for s in range(n_steps):
        jl = 1 << s
        mats.append(np.roll(eye, -jl, axis=1))
        mats.append(np.roll(eye, jl, axis=1))
    return np.stack(mats)


def _bitonic_sort(v, fm_ref, pm_ref):
    grps = v.shape[1] * v.shape[2]
    n = grps * LANES
    log_n = n.bit_length() - 1
    w = _flip(v, fm_ref[0][None])
    for stage in range(1, log_n + 1):
        k = 1 << stage
        j = k >> 1
        while j >= 1:
            w = _cmp_exchange_asc(w, j, pm_ref)
            j >>= 1
        if k < n:
            w = _flip(w, fm_ref[stage][None])
    return w


def _loss_kernel(x_ref, y_ref, fm_ref, pm_ref, out_ref, *, r_block, total, ch):
    i = pl.program_id(0)

    @pl.when(i == 0)
    def _init():
        out_ref[0, 0] = 0.0

    def body(r, acc):
        xs = x_ref[pl.ds(r * ch, ch)]
        ys = y_ref[pl.ds(r * ch, ch)]
        v = jnp.concatenate([xs, ys], axis=0)
        v = _bitonic_sort(v, fm_ref, pm_ref)
        d = v[:ch] - v[ch:]
        return acc + jnp.sum(d * d)

    acc = jax.lax.fori_loop(0, r_block // ch, body, jnp.float32(0.0))
    out_ref[0, 0] += acc

    @pl.when(i == pl.num_programs(0) - 1)
    def _finish():
        out_ref[0, 0] = out_ref[0, 0] / total


@jax.jit
def kernel(x, y):
    rows, n = x.shape
    grps = n // LANES
    r_block = 128
    ch = 16
    log_n = n.bit_length() - 1
    inner = min(grps, 8)
    outer = grps // inner
    x3 = x.reshape(rows, outer, inner, LANES)
    y3 = y.reshape(rows, outer, inner, LANES)
    fm = jnp.asarray(_flip_masks(grps))
    pm = jnp.asarray(_perm_mats(grps, n))
    pm_shape = pm.shape
    out = pl.pallas_call(
        functools.partial(_loss_kernel, r_block=r_block, total=rows * n,
                          ch=ch),
        grid=(rows // r_block,),
        in_specs=[
            pl.BlockSpec((r_block, outer, inner, LANES),
                         lambda i: (i, 0, 0, 0)),
            pl.BlockSpec((r_block, outer, inner, LANES),
                         lambda i: (i, 0, 0, 0)),
            pl.BlockSpec((log_n, outer, inner, LANES),
                         lambda i: (0, 0, 0, 0)),
            pl.BlockSpec(pm_shape, lambda i: (0, 0, 0)),
        ],
        out_specs=pl.BlockSpec((1, 1), lambda i: (0, 0),
                               memory_space=pltpu.SMEM),
        out_shape=jax.ShapeDtypeStruct((1, 1), jnp.float32),
    )(x3, y3, fm, pm)
    return out[0, 0]


# speedup vs baseline: 6.3661x; 6.3661x over previous
import functools

import jax
import jax.numpy as jnp
import numpy as np
from jax.experimental import pallas as pl
from jax.experimental.pallas import tpu as pltpu

LANES = 128


def _roll(v, shift, axis):
    return pltpu.roll(v, shift % v.shape[axis], axis)


def _flip(w, m):
    return jax.lax.bitcast_convert_type(
        jax.lax.bitcast_convert_type(w, jnp.int32) ^ m, jnp.float32)


def _lane_roll_mxu(w, pmat):
    b, o, i, l = w.shape
    w2 = w.reshape(b * o * i, l)
    r = jax.lax.dot_general(w2, pmat, (((1,), (0,)), ((), ())),
                            preferred_element_type=jnp.float32)
    return r.reshape(b, o, i, l)


def _cmp_exchange_asc(w, j, pm_ref):
    outer, inner = w.shape[1], w.shape[2]
    grps = outer * inner
    if j < inner:
        axis, sh = 2, j
        it = jax.lax.broadcasted_iota(jnp.int32, (1, 1, inner, 1), 2)
    elif j < grps:
        axis, sh = 1, j // inner
        it = jax.lax.broadcasted_iota(jnp.int32, (1, outer, 1, 1), 1)
    else:
        sh = j // grps
        it = jax.lax.broadcasted_iota(jnp.int32, (1, 1, 1, LANES), 3)
        left = (it & sh) == 0
        s = sh.bit_length() - 1
        b = _lane_roll_mxu(w, pm_ref[2 * s])
        mn = jnp.minimum(w, b)
        mx = jnp.maximum(w, b)
        mxr = _lane_roll_mxu(mx, pm_ref[2 * s + 1])
        return jnp.where(left, mn, mxr)
    left = (it & sh) == 0
    b = _roll(w, -sh, axis)
    mn = jnp.minimum(w, b)
    mx = jnp.maximum(w, b)
    mxr = _roll(mx, sh, axis)
    return jnp.where(left, mn, mxr)


def _flip_masks(grps):
    n = grps * LANES
    log_n = n.bit_length() - 1
    li = np.arange(LANES, dtype=np.int64)[None, :]
    gi = np.arange(grps, dtype=np.int64)[:, None]
    idxn = li * grps + gi
    neg = np.int32(-2 ** 31)

    def smask(k):
        return np.where((idxn & k) != 0, neg, np.int32(0)).astype(np.int32)

    out = [smask(2)]
    for stage in range(1, log_n):
        k = 1 << stage
        out.append(smask(k) ^ smask(2 * k))
    inner = min(grps, 8)
    return np.stack(out).reshape(log_n, grps // inner, inner, LANES)


def _perm_mats(grps, n):
    n_steps = (n // grps).bit_length() - 1
    eye = np.eye(LANES, dtype=np.float32)
    mats = []
    for s in range(n_steps):
        jl = 1 << s
        mats.append(np.roll(eye, -jl, axis=1))
        mats.append(np.roll(eye, jl, axis=1))
    return np.stack(mats)


def _bitonic_sort(v, fm_ref, pm_ref):
    grps = v.shape[1] * v.shape[2]
    n = grps * LANES
    log_n = n.bit_length() - 1
    w = _flip(v, fm_ref[0][None])
    for stage in range(1, log_n + 1):
        k = 1 << stage
        j = k >> 1
        while j >= 1:
            w = _cmp_exchange_asc(w, j, pm_ref)
            j >>= 1
        if k < n:
            w = _flip(w, fm_ref[stage][None])
    return w


def _loss_kernel(x_ref, y_ref, fm_ref, pm_ref, out_ref, *, r_block, total, ch):
    def body(r, acc):
        xs = x_ref[pl.ds(r * ch, ch)]
        ys = y_ref[pl.ds(r * ch, ch)]
        v = jnp.concatenate([xs, ys], axis=0)
        v = _bitonic_sort(v, fm_ref, pm_ref)
        d = v[:ch] - v[ch:]
        return acc + jnp.sum(d * d)

    acc = jax.lax.fori_loop(0, r_block // ch, body, jnp.float32(0.0))
    out_ref[0, 0, 0] = acc / total


@jax.jit
def kernel(x, y):
    rows, n = x.shape
    grps = n // LANES
    r_block = 128
    ch = 16
    log_n = n.bit_length() - 1
    inner = min(grps, 8)
    outer = grps // inner
    x3 = x.reshape(rows, outer, inner, LANES)
    y3 = y.reshape(rows, outer, inner, LANES)
    fm = jnp.asarray(_flip_masks(grps))
    pm = jnp.asarray(_perm_mats(grps, n))
    pm_shape = pm.shape
    out = pl.pallas_call(
        functools.partial(_loss_kernel, r_block=r_block, total=rows * n,
                          ch=ch),
        grid=(rows // r_block,),
        in_specs=[
            pl.BlockSpec((r_block, outer, inner, LANES),
                         lambda i: (i, 0, 0, 0)),
            pl.BlockSpec((r_block, outer, inner, LANES),
                         lambda i: (i, 0, 0, 0)),
            pl.BlockSpec((log_n, outer, inner, LANES),
                         lambda i: (0, 0, 0, 0)),
            pl.BlockSpec(pm_shape, lambda i: (0, 0, 0)),
        ],
        out_specs=pl.BlockSpec((1, 1, 1), lambda i: (i, 0, 0),
                               memory_space=pltpu.SMEM),
        out_shape=jax.ShapeDtypeStruct((rows // r_block, 1, 1), jnp.float32),
        compiler_params=pltpu.CompilerParams(
            dimension_semantics=("parallel",)),
    )(x3, y3, fm, pm)
    return jnp.sum(out[:, 0, 0])


# speedup vs baseline: 7.0310x; 1.1045x over previous
import functools

import jax
import jax.numpy as jnp
import numpy as np
from jax.experimental import pallas as pl
from jax.experimental.pallas import tpu as pltpu

LANES = 128


def _roll(v, shift, axis):
    return pltpu.roll(v, shift % v.shape[axis], axis)


def _flip(w, m):
    return jax.lax.bitcast_convert_type(
        jax.lax.bitcast_convert_type(w, jnp.int32) ^ m, jnp.float32)


def _lane_roll_mxu(w, pmat):
    b, o, i, l = w.shape
    w2 = w.reshape(b * o * i, l)
    r = jax.lax.dot_general(w2, pmat, (((1,), (0,)), ((), ())),
                            preferred_element_type=jnp.float32)
    return r.reshape(b, o, i, l)


def _cmp_exchange_asc(w, j, pm_ref):
    outer, inner = w.shape[1], w.shape[2]
    grps = outer * inner
    if j < outer:
        axis, sh = 1, j
        it = jax.lax.broadcasted_iota(jnp.int32, (1, outer, 1, 1), 1)
    elif j < grps:
        axis, sh = 2, j // outer
        it = jax.lax.broadcasted_iota(jnp.int32, (1, 1, inner, 1), 2)
    else:
        sh = j // grps
        it = jax.lax.broadcasted_iota(jnp.int32, (1, 1, 1, LANES), 3)
        left = (it & sh) == 0
        s = sh.bit_length() - 1
        b = _lane_roll_mxu(w, pm_ref[2 * s])
        mn = jnp.minimum(w, b)
        mx = jnp.maximum(w, b)
        mxr = _lane_roll_mxu(mx, pm_ref[2 * s + 1])
        return jnp.where(left, mn, mxr)
    left = (it & sh) == 0
    b = _roll(w, -sh, axis)
    mn = jnp.minimum(w, b)
    mx = jnp.maximum(w, b)
    mxr = _roll(mx, sh, axis)
    return jnp.where(left, mn, mxr)


def _flip_masks(grps):
    n = grps * LANES
    log_n = n.bit_length() - 1
    inner = min(grps, 8)
    outer = grps // inner
    li = np.arange(LANES, dtype=np.int64)[None, None, :]
    ii = np.arange(inner, dtype=np.int64)[None, :, None]
    oi = np.arange(outer, dtype=np.int64)[:, None, None]
    idxn = oi + outer * ii + grps * li
    neg = np.int32(-2 ** 31)

    def smask(k):
        return np.where((idxn & k) != 0, neg, np.int32(0)).astype(np.int32)

    out = [smask(2)]
    for stage in range(1, log_n):
        k = 1 << stage
        out.append(smask(k) ^ smask(2 * k))
    return np.stack(out)


def _perm_mats(grps, n):
    n_steps = (n // grps).bit_length() - 1
    eye = np.eye(LANES, dtype=np.float32)
    mats = []
    for s in range(n_steps):
        jl = 1 << s
        mats.append(np.roll(eye, -jl, axis=1))
        mats.append(np.roll(eye, jl, axis=1))
    return np.stack(mats)


def _bitonic_sort(v, fm_ref, pm_ref):
    grps = v.shape[1] * v.shape[2]
    n = grps * LANES
    log_n = n.bit_length() - 1
    w = _flip(v, fm_ref[0][None])
    for stage in range(1, log_n + 1):
        k = 1 << stage
        j = k >> 1
        while j >= 1:
            w = _cmp_exchange_asc(w, j, pm_ref)
            j >>= 1
        if k < n:
            w = _flip(w, fm_ref[stage][None])
    return w


def _loss_kernel(x_ref, y_ref, fm_ref, pm_ref, out_ref, *, r_block, total, ch):
    def body(r, acc):
        xs = x_ref[pl.ds(r * ch, ch)]
        ys = y_ref[pl.ds(r * ch, ch)]
        v = jnp.concatenate([xs, ys], axis=0)
        v = _bitonic_sort(v, fm_ref, pm_ref)
        d = v[:ch] - v[ch:]
        return acc + jnp.sum(d * d)

    acc = jax.lax.fori_loop(0, r_block // ch, body, jnp.float32(0.0))
    out_ref[0, 0, 0] = acc / total


@jax.jit
def kernel(x, y):
    rows, n = x.shape
    grps = n // LANES
    r_block = 128
    ch = 16
    log_n = n.bit_length() - 1
    inner = min(grps, 8)
    outer = grps // inner
    x3 = x.reshape(rows, outer, inner, LANES)
    y3 = y.reshape(rows, outer, inner, LANES)
    fm = jnp.asarray(_flip_masks(grps))
    pm = jnp.asarray(_perm_mats(grps, n))
    pm_shape = pm.shape
    out = pl.pallas_call(
        functools.partial(_loss_kernel, r_block=r_block, total=rows * n,
                          ch=ch),
        grid=(rows // r_block,),
        in_specs=[
            pl.BlockSpec((r_block, outer, inner, LANES),
                         lambda i: (i, 0, 0, 0)),
            pl.BlockSpec((r_block, outer, inner, LANES),
                         lambda i: (i, 0, 0, 0)),
            pl.BlockSpec((log_n, outer, inner, LANES),
                         lambda i: (0, 0, 0, 0)),
            pl.BlockSpec(pm_shape, lambda i: (0, 0, 0)),
        ],
        out_specs=pl.BlockSpec((1, 1, 1), lambda i: (i, 0, 0),
                               memory_space=pltpu.SMEM),
        out_shape=jax.ShapeDtypeStruct((rows // r_block, 1, 1), jnp.float32),
        compiler_params=pltpu.CompilerParams(
            dimension_semantics=("parallel",)),
    )(x3, y3, fm, pm)
    return jnp.sum(out[:, 0, 0])
